# baseline (speedup 1.0000x reference)
import functools

import jax
import jax.numpy as jnp
from jax import lax
from jax.experimental import pallas as pl
from jax.experimental.pallas import tpu as pltpu
from jax.experimental.pallas import tpu_sc as plsc

N = 16384
D = 256
L = 16
NW = 32
COLS_PER_W = D // NW
C = N // L
U = 4
CS = C // U
NBINS = 256
INV = 1.0 / (N + 1)

_mesh = plsc.VectorSubcoreMesh(core_axis_name="c", subcore_axis_name="s")


@functools.partial(
    pl.kernel,
    out_type=jax.ShapeDtypeStruct((1, D, N), jnp.float32),
    mesh=_mesh,
    scratch_types=[
        pltpu.VMEM((N,), jnp.float32),
        pltpu.VMEM((N,), jnp.int32),
        pltpu.VMEM((N,), jnp.int32),
        pltpu.VMEM((N,), jnp.int32),
        pltpu.VMEM((N,), jnp.int32),
        pltpu.VMEM((N,), jnp.float32),
        pltpu.VMEM((NBINS * L,), jnp.int32),
        pltpu.VMEM((NBINS * L,), jnp.int32),
        pltpu.VMEM((NBINS * L,), jnp.int32),
        pltpu.VMEM((NBINS * L,), jnp.int32),
    ],
    compiler_params=pltpu.CompilerParams(needs_layout_passes=False),
)
def _rank_all_columns(x_hbm, out_hbm, inbuf, key_a, key_b, pay_a, pay_b,
                      outbuf, hist0, hist1, hist2, hist3):
    wid = lax.axis_index("s") * 2 + lax.axis_index("c")
    lanes = lax.iota(jnp.int32, L)
    lane_base = lanes * C
    ones = jnp.ones((L,), jnp.int32)
    zeros = jnp.zeros((L,), jnp.int32)
    hists = (hist0, hist1, hist2, hist3)

    def to_key(xf):
        xi = lax.bitcast_convert_type(xf + 0.0, jnp.int32)
        m = lax.shift_right_arithmetic(xi, 31)
        return lax.bitwise_xor(xi, lax.bitwise_or(m, jnp.int32(-(2**31))))

    def slot_of(key, sh):
        d = lax.bitwise_and(lax.shift_right_logical(key, sh), jnp.int32(0xFF))
        return lax.shift_left(d, jnp.int32(4)) + lanes

    def radix_pass(sh, get_key, get_pay, emit):
        def zf(i, carry):
            ds_ = pl.ds(i * L, L)
            for h in hists:
                h[ds_] = zeros
            return carry
        lax.fori_loop(0, NBINS, zf, 0, unroll=4)

        def histf(t, carry):
            for u in range(U):
                k = get_key(u, t)
                plsc.addupdate_scatter(hists[u], [slot_of(k, sh)], ones)
            return carry
        lax.fori_loop(0, CS, histf, 0)

        def sf(i, carry):
            ds_ = pl.ds(i * L, L)
            v0 = hist0[ds_]
            v1 = hist1[ds_]
            v2 = hist2[ds_]
            v3 = hist3[ds_]
            t01 = v0 + v1
            tot = t01 + v2 + v3
            inc = plsc.cumsum(tot)
            base = inc - tot + carry
            hist0[ds_] = base
            hist1[ds_] = base + v0
            hist2[ds_] = base + t01
            hist3[ds_] = base + t01 + v2
            return carry + jnp.sum(tot)
        lax.fori_loop(0, NBINS, sf, jnp.int32(0), unroll=4)

        def permf(t, carry):
            for u in range(U):
                k = get_key(u, t)
                s = slot_of(k, sh)
                p = plsc.load_gather(hists[u], [s])
                plsc.addupdate_scatter(hists[u], [s], ones)
                emit(p, k, get_pay(u, t))
            return carry
        lax.fori_loop(0, CS, permf, 0)

    def do_col(ci, carry):
        col = wid * COLS_PER_W + ci
        pltpu.sync_copy(x_hbm.at[col], inbuf)

        def k_in(u, t):
            return to_key(plsc.load_gather(inbuf, [lane_base + (u * CS) + t]))

        def p_iota(u, t):
            return lane_base + (u * CS) + t

        def emit_ab(p, k, pv):
            plsc.store_scatter(key_a, [p], k)
            plsc.store_scatter(pay_a, [p], pv)

        radix_pass(0, k_in, p_iota, emit_ab)

        def k_a(u, t):
            return plsc.load_gather(key_a, [lane_base + (u * CS) + t])

        def p_a(u, t):
            return plsc.load_gather(pay_a, [lane_base + (u * CS) + t])

        def emit_ba(p, k, pv):
            plsc.store_scatter(key_b, [p], k)
            plsc.store_scatter(pay_b, [p], pv)

        radix_pass(8, k_a, p_a, emit_ba)

        def k_b(u, t):
            return plsc.load_gather(key_b, [lane_base + (u * CS) + t])

        def p_b(u, t):
            return plsc.load_gather(pay_b, [lane_base + (u * CS) + t])

        radix_pass(16, k_b, p_b, emit_ab)

        def emit_out(p, k, pv):
            val = lax.convert_element_type(p + 1, jnp.float32) * INV
            plsc.store_scatter(outbuf, [pv], val)

        radix_pass(24, k_a, p_a, emit_out)

        pltpu.sync_copy(outbuf, out_hbm.at[0, col])
        return carry

    lax.fori_loop(0, COLS_PER_W, do_col, 0)


def kernel(samples):
    xt = jnp.transpose(samples)
    return _rank_all_columns(xt)


# speedup vs baseline: 1.1424x; 1.1424x over previous
import functools

import jax
import jax.numpy as jnp
from jax import lax
from jax.experimental import pallas as pl
from jax.experimental.pallas import tpu as pltpu
from jax.experimental.pallas import tpu_sc as plsc

N = 16384
D = 256
L = 16
NW = 32
COLS_PER_W = D // NW
C = N // L
W = 4
NBINS = 256
INV = 1.0 / (N + 1)

_mesh = plsc.VectorSubcoreMesh(core_axis_name="c", subcore_axis_name="s")


@functools.partial(
    pl.kernel,
    out_type=jax.ShapeDtypeStruct((1, D, N), jnp.float32),
    mesh=_mesh,
    scratch_types=[
        pltpu.VMEM((N,), jnp.float32),
        pltpu.VMEM((N,), jnp.int32),
        pltpu.VMEM((N,), jnp.int32),
        pltpu.VMEM((N,), jnp.int32),
        pltpu.VMEM((N,), jnp.int32),
        pltpu.VMEM((N,), jnp.float32),
        pltpu.VMEM((N,), jnp.int32),
        pltpu.VMEM((NBINS * L,), jnp.int32),
        pltpu.SMEM((NBINS,), jnp.int32),
    ],
    compiler_params=pltpu.CompilerParams(needs_layout_passes=False),
)
def _rank_all_columns(x_hbm, out_hbm, inbuf, key_a, key_b, pay_a, pay_b,
                      outbuf, sbuf, hist, tbuf):
    wid = lax.axis_index("s") * 2 + lax.axis_index("c")
    lanes = lax.iota(jnp.int32, L)
    lane_base = lanes * C
    ones = jnp.ones((L,), jnp.int32)
    zeros = jnp.zeros((L,), jnp.int32)

    def to_key(xf):
        xi = lax.bitcast_convert_type(xf + 0.0, jnp.int32)
        m = lax.shift_right_arithmetic(xi, 31)
        return lax.bitwise_xor(xi, lax.bitwise_or(m, jnp.int32(-(2**31))))

    def slot_of(key, sh):
        d = lax.bitwise_and(lax.shift_right_logical(key, sh), jnp.int32(0xFF))
        return lax.shift_left(d, jnp.int32(4)) + lanes

    def radix_pass(sh, get_key, get_pay, emit):
        @plsc.parallel_loop(0, NBINS, unroll=4)
        def _zero(i):
            hist[pl.ds(i * L, L)] = zeros

        @plsc.parallel_loop(0, C, unroll=4)
        def _hist(t):
            s = slot_of(get_key(t), sh)
            plsc.addupdate_scatter(hist, [s], ones)
            plsc.store_scatter(sbuf, [lane_base + t], s)

        @plsc.parallel_loop(0, NBINS, unroll=4)
        def _scan1(d):
            v = hist[pl.ds(d * L, L)]
            inc = plsc.cumsum(v)
            hist[pl.ds(d * L, L)] = inc - v
            tbuf[d] = jnp.sum(v)

        def _scan2(i, carry):
            v = tbuf[i]
            tbuf[i] = carry
            return carry + v
        lax.fori_loop(0, NBINS, _scan2, jnp.int32(0), unroll=4)

        @plsc.parallel_loop(0, NBINS, unroll=4)
        def _scan3(d):
            hist[pl.ds(d * L, L)] = hist[pl.ds(d * L, L)] + tbuf[d]

        def _pos(w, carry):
            t0 = w * W
            idx = [lane_base + (t0 + j) for j in range(W)]
            s = [plsc.load_gather(sbuf, [idx[j]]) for j in range(W)]
            r = [plsc.load_gather(hist, [s[j]]) for j in range(W)]
            o = [r[0]]
            for j in range(1, W):
                acc = r[j]
                for jp in range(j):
                    acc = acc + jnp.where(s[j] == s[jp], 1, 0).astype(jnp.int32)
                o.append(acc)
            for j in range(W):
                plsc.store_scatter(sbuf, [idx[j]], o[j])
            for j in range(W):
                plsc.addupdate_scatter(hist, [s[j]], ones)
            return carry
        lax.fori_loop(0, C // W, _pos, 0)

        @plsc.parallel_loop(0, C, unroll=4)
        def _perm(t):
            p = plsc.load_gather(sbuf, [lane_base + t])
            emit(p, get_key, get_pay, t)

    def do_col(ci, carry):
        col = wid * COLS_PER_W + ci
        pltpu.sync_copy(x_hbm.at[col], inbuf)

        def k_in(t):
            return to_key(plsc.load_gather(inbuf, [lane_base + t]))

        def p_iota(t):
            return lane_base + t

        def emit_ab(p, gk, gp, t):
            plsc.store_scatter(key_a, [p], gk(t))
            plsc.store_scatter(pay_a, [p], gp(t))

        radix_pass(0, k_in, p_iota, emit_ab)

        def k_a(t):
            return plsc.load_gather(key_a, [lane_base + t])

        def p_a(t):
            return plsc.load_gather(pay_a, [lane_base + t])

        def emit_ba(p, gk, gp, t):
            plsc.store_scatter(key_b, [p], gk(t))
            plsc.store_scatter(pay_b, [p], gp(t))

        radix_pass(8, k_a, p_a, emit_ba)

        def k_b(t):
            return plsc.load_gather(key_b, [lane_base + t])

        def p_b(t):
            return plsc.load_gather(pay_b, [lane_base + t])

        radix_pass(16, k_b, p_b, emit_ab)

        def emit_out(p, gk, gp, t):
            val = lax.convert_element_type(p + 1, jnp.float32) * INV
            plsc.store_scatter(outbuf, [gp(t)], val)

        radix_pass(24, k_a, p_a, emit_out)

        pltpu.sync_copy(outbuf, out_hbm.at[0, col])
        return carry

    lax.fori_loop(0, COLS_PER_W, do_col, 0)


def kernel(samples):
    xt = jnp.transpose(samples)
    return _rank_all_columns(xt)
